# Initial kernel scaffold; baseline (speedup 1.0000x reference)
import functools

import jax
import jax.numpy as jnp
from jax import lax
from jax.experimental import pallas as pl
from jax.experimental.pallas import tpu as pltpu
from jax.experimental.pallas import tpu_sc as plsc

N = 10000
E = 320000
D = 128
G = 64

NC = 2
NS = 16
NW = NC * NS
CHUNK = 128
EPW = -(-E // (NW * CHUNK))
E_PAD = NW * EPW * CHUNK
RPT = 640
N_PAD = NS * RPT


def _sc_degree(dsts):
    mesh = plsc.VectorSubcoreMesh(core_axis_name="c", subcore_axis_name="s")

    @functools.partial(
        pl.kernel,
        mesh=mesh,
        out_type=jax.ShapeDtypeStruct((NC, N_PAD), jnp.float32),
        scratch_types=[
            pltpu.VMEM((EPW, CHUNK), jnp.int32),
            pltpu.VMEM((CHUNK,), jnp.float32),
            pltpu.VMEM((RPT,), jnp.float32),
            pltpu.VMEM_SHARED((N_PAD,), jnp.float32),
        ],
    )
    def deg_kernel(dsts_hbm, out_hbm, idx_v, ones_v, z_v, acc_s):
        c = lax.axis_index("c")
        s = lax.axis_index("s")
        w = c * NS + s
        pltpu.sync_copy(dsts_hbm.at[w], idx_v)
        for i in range(CHUNK // 16):
            ones_v[pl.ds(i * 16, 16)] = jnp.ones((16,), jnp.float32)
        for i in range(RPT // 16):
            z_v[pl.ds(i * 16, 16)] = jnp.zeros((16,), jnp.float32)
        pltpu.sync_copy(z_v, acc_s.at[pl.ds(s * RPT, RPT)])
        plsc.subcore_barrier()

        def body(j, carry):
            pltpu.sync_copy(ones_v, acc_s.at[idx_v.at[j]], add=True)
            return carry

        lax.fori_loop(0, EPW, body, 0)
        plsc.subcore_barrier()
        pltpu.sync_copy(acc_s.at[pl.ds(s * RPT, RPT)],
                        out_hbm.at[c, pl.ds(s * RPT, RPT)])

    return deg_kernel(dsts)


def _sc_aggregate(hs, srcs, dsts):
    mesh = plsc.VectorSubcoreMesh(core_axis_name="c", subcore_axis_name="s")

    @functools.partial(
        pl.kernel,
        mesh=mesh,
        out_type=jax.ShapeDtypeStruct((NC, N_PAD, D), jnp.float32),
        scratch_types=[
            pltpu.VMEM((EPW, CHUNK), jnp.int32),
            pltpu.VMEM((EPW, CHUNK), jnp.int32),
            pltpu.VMEM((CHUNK, D), jnp.float32),
            pltpu.VMEM((CHUNK, D), jnp.float32),
            pltpu.VMEM_SHARED((N_PAD, D), jnp.float32),
            pltpu.SemaphoreType.DMA,
        ],
    )
    def agg_kernel(hs_hbm, srcs_hbm, dsts_hbm, out_hbm,
                   src_v, dst_v, buf_v, z_v, acc_s, sem):
        c = lax.axis_index("c")
        s = lax.axis_index("s")
        w = c * NS + s
        pltpu.sync_copy(srcs_hbm.at[w], src_v)
        pltpu.sync_copy(dsts_hbm.at[w], dst_v)

        def zrow(r, carry):
            for i in range(D // 16):
                z_v[r, pl.ds(i * 16, 16)] = jnp.zeros((16,), jnp.float32)
            return carry

        lax.fori_loop(0, CHUNK, zrow, 0)
        for t in range(RPT // CHUNK):
            pltpu.sync_copy(z_v, acc_s.at[pl.ds(s * RPT + t * CHUNK, CHUNK)])
        plsc.subcore_barrier()

        def body(j, carry):
            pltpu.async_copy(hs_hbm.at[src_v.at[j]], buf_v, sem).wait()
            pltpu.sync_copy(buf_v, acc_s.at[dst_v.at[j]], add=True)
            return carry

        lax.fori_loop(0, EPW, body, 0)
        plsc.subcore_barrier()
        pltpu.sync_copy(acc_s.at[pl.ds(s * RPT, RPT)],
                        out_hbm.at[c, pl.ds(s * RPT, RPT)])

    return agg_kernel(hs, srcs, dsts)


def _tc_scale_matmul(x, W1, deg_col):
    def body(x_ref, w_ref, deg_ref, out_ref):
        dinv = lax.rsqrt(deg_ref[...] + 1.0)
        h = jnp.dot(x_ref[...], w_ref[...], preferred_element_type=jnp.float32)
        out_ref[...] = h * dinv

    return pl.pallas_call(
        body,
        out_shape=jax.ShapeDtypeStruct((N, D), jnp.float32),
    )(x, W1, deg_col)


def _tc_mid(deg_col, p1, hs1, b1, gamma, beta, run_mean, run_var, W2):
    def body(deg_ref, p_ref, hs_ref, b1_ref, g_ref, be_ref, rm_ref, rv_ref,
             w_ref, out_ref):
        dinv = lax.rsqrt(deg_ref[...] + 1.0)
        agg = p_ref[0, :N, :] + p_ref[1, :N, :] + hs_ref[...]
        out1 = agg * dinv + b1_ref[...]
        scale = g_ref[...] * lax.rsqrt(rv_ref[...] + 1e-5)
        h = (out1 - rm_ref[...]) * scale + be_ref[...]
        h = jnp.maximum(h, 0.0)
        h2 = jnp.dot(h, w_ref[...], preferred_element_type=jnp.float32)
        out_ref[...] = h2 * dinv

    return pl.pallas_call(
        body,
        out_shape=jax.ShapeDtypeStruct((N, D), jnp.float32),
    )(deg_col, p1, hs1, b1, gamma, beta, run_mean, run_var, W2)


def _tc_head(deg_col, p2, hs2, b2, batch_row, fc1_W, fc1_b, cls_W, cls_b):
    def body(deg_ref, p_ref, hs_ref, b2_ref, batch_ref, fw_ref, fb_ref,
             cw_ref, cb_ref, out_ref):
        dinv = lax.rsqrt(deg_ref[...] + 1.0)
        agg = p_ref[0, :N, :] + p_ref[1, :N, :] + hs_ref[...]
        out2 = agg * dinv + b2_ref[...]
        seg = lax.broadcasted_iota(jnp.int32, (G, N), 0)
        onehot = (seg == batch_ref[...]).astype(jnp.float32)
        sums = jnp.dot(onehot, out2, preferred_element_type=jnp.float32)
        cnt = jnp.sum(onehot, axis=1, keepdims=True)
        pooled = sums / jnp.maximum(cnt, 1.0)
        h = jnp.dot(pooled, fw_ref[...], preferred_element_type=jnp.float32)
        h = jnp.maximum(h + fb_ref[...], 0.0)
        out_ref[...] = jnp.dot(h, cw_ref[...],
                               preferred_element_type=jnp.float32) + cb_ref[...]

    return pl.pallas_call(
        body,
        out_shape=jax.ShapeDtypeStruct((G, 2), jnp.float32),
    )(deg_col, p2, hs2, b2, batch_row, fc1_W, fc1_b, cls_W, cls_b)


def kernel(x, edge_index, batch, W1, b1, gamma, beta, run_mean, run_var,
           W2, b2, fc1_W, fc1_b, cls_W, cls_b):
    src = edge_index[0]
    dst = edge_index[1]
    pad = E_PAD - E
    srcs = jnp.concatenate([src, jnp.zeros((pad,), src.dtype)])
    dsts = jnp.concatenate([dst, jnp.full((pad,), N, dst.dtype)])
    srcs = srcs.reshape(NW, EPW, CHUNK)
    dsts = dsts.reshape(NW, EPW, CHUNK)

    deg_parts = _sc_degree(dsts)
    deg_col = (deg_parts[0, :N] + deg_parts[1, :N]).reshape(N, 1)

    hs1 = _tc_scale_matmul(x, W1, deg_col)
    p1 = _sc_aggregate(hs1, srcs, dsts)
    hs2 = _tc_mid(deg_col, p1, hs1, b1.reshape(1, D), gamma.reshape(1, D),
                  beta.reshape(1, D), run_mean.reshape(1, D),
                  run_var.reshape(1, D), W2)
    p2 = _sc_aggregate(hs2, srcs, dsts)
    out = _tc_head(deg_col, p2, hs2, b2.reshape(1, D), batch.reshape(1, N),
                   fc1_W, fc1_b.reshape(1, D // 2), cls_W, cls_b.reshape(1, 2))
    return out


# speedup vs baseline: 13.4912x; 13.4912x over previous
import functools

import jax
import jax.numpy as jnp
from jax import lax
from jax.experimental import pallas as pl
from jax.experimental.pallas import tpu as pltpu
from jax.experimental.pallas import tpu_sc as plsc

N = 10000
E = 320000
D = 128
G = 64

NC = 2
NS = 16
NW = NC * NS
CHUNK = 128
EPW = -(-E // (NW * CHUNK))
E_PAD = NW * EPW * CHUNK
RPT = 640
N_PAD = NS * RPT
SHIFT = 14
MASK = (1 << SHIFT) - 1


def _sc_degree(packed):
    mesh = plsc.VectorSubcoreMesh(core_axis_name="c", subcore_axis_name="s")

    @functools.partial(
        pl.kernel,
        mesh=mesh,
        out_type=jax.ShapeDtypeStruct((NC, N_PAD), jnp.float32),
        scratch_types=[
            pltpu.VMEM((EPW, CHUNK), jnp.int32),
            pltpu.VMEM((EPW, CHUNK), jnp.int32),
            pltpu.VMEM((CHUNK,), jnp.float32),
            pltpu.VMEM((RPT,), jnp.float32),
            pltpu.VMEM_SHARED((N_PAD,), jnp.float32),
        ],
    )
    def deg_kernel(pk_hbm, out_hbm, pk_v, dst_v, ones_v, z_v, acc_s):
        c = lax.axis_index("c")
        s = lax.axis_index("s")
        w = c * NS + s
        pltpu.sync_copy(pk_hbm.at[w], pk_v)
        for i in range(CHUNK // 16):
            ones_v[pl.ds(i * 16, 16)] = jnp.ones((16,), jnp.float32)
        for i in range(RPT // 16):
            z_v[pl.ds(i * 16, 16)] = jnp.zeros((16,), jnp.float32)
        pltpu.sync_copy(z_v, acc_s.at[pl.ds(s * RPT, RPT)])

        def unpack(r, carry):
            for i in range(CHUNK // 16):
                v = pk_v[r, pl.ds(i * 16, 16)]
                dst_v[r, pl.ds(i * 16, 16)] = jnp.bitwise_and(v, MASK)
            return carry

        lax.fori_loop(0, EPW, unpack, 0)
        plsc.subcore_barrier()

        def body(j, carry):
            pltpu.sync_copy(ones_v, acc_s.at[dst_v.at[j]], add=True)
            return carry

        lax.fori_loop(0, EPW, body, 0)
        plsc.subcore_barrier()
        pltpu.sync_copy(acc_s.at[pl.ds(s * RPT, RPT)],
                        out_hbm.at[c, pl.ds(s * RPT, RPT)])

    return deg_kernel(packed)


def _sc_aggregate(hs, packed):
    mesh = plsc.VectorSubcoreMesh(core_axis_name="c", subcore_axis_name="s")

    @functools.partial(
        pl.kernel,
        mesh=mesh,
        out_type=jax.ShapeDtypeStruct((NC, N_PAD, D), jnp.float32),
        scratch_types=[
            pltpu.VMEM((EPW, CHUNK), jnp.int32),
            pltpu.VMEM((EPW, CHUNK), jnp.int32),
            pltpu.VMEM((CHUNK, D), jnp.float32),
            pltpu.VMEM_SHARED((N_PAD, D), jnp.float32),
            pltpu.SemaphoreType.DMA,
        ],
    )
    def agg_kernel(hs_hbm, pk_hbm, out_hbm,
                   pk_v, src_v, buf_v, acc_s, sem):
        c = lax.axis_index("c")
        s = lax.axis_index("s")
        w = c * NS + s
        pltpu.sync_copy(pk_hbm.at[w], pk_v)

        def zrow(r, carry):
            for i in range(D // 16):
                buf_v[r, pl.ds(i * 16, 16)] = jnp.zeros((16,), jnp.float32)
            return carry

        lax.fori_loop(0, CHUNK, zrow, 0)
        for t in range(RPT // CHUNK):
            pltpu.sync_copy(buf_v, acc_s.at[pl.ds(s * RPT + t * CHUNK, CHUNK)])

        def unpack(r, carry):
            for i in range(CHUNK // 16):
                v = pk_v[r, pl.ds(i * 16, 16)]
                src_v[r, pl.ds(i * 16, 16)] = jnp.right_shift(v, SHIFT)
                pk_v[r, pl.ds(i * 16, 16)] = jnp.bitwise_and(v, MASK)
            return carry

        lax.fori_loop(0, EPW, unpack, 0)
        plsc.subcore_barrier()

        def body(j, carry):
            pltpu.async_copy(hs_hbm.at[src_v.at[j]], buf_v, sem).wait()
            pltpu.sync_copy(buf_v, acc_s.at[pk_v.at[j]], add=True)
            return carry

        lax.fori_loop(0, EPW, body, 0)
        plsc.subcore_barrier()
        pltpu.sync_copy(acc_s.at[pl.ds(s * RPT, RPT)],
                        out_hbm.at[c, pl.ds(s * RPT, RPT)])

    return agg_kernel(hs, packed)


def _tc_scale_matmul(x, W1, deg_col):
    def body(x_ref, w_ref, deg_ref, out_ref):
        dinv = lax.rsqrt(deg_ref[...] + 1.0)
        h = jnp.dot(x_ref[...], w_ref[...], preferred_element_type=jnp.float32)
        out_ref[...] = h * dinv

    return pl.pallas_call(
        body,
        out_shape=jax.ShapeDtypeStruct((N, D), jnp.float32),
    )(x, W1, deg_col)


def _tc_mid(deg_col, p1, hs1, b1, gamma, beta, run_mean, run_var, W2):
    def body(deg_ref, p_ref, hs_ref, b1_ref, g_ref, be_ref, rm_ref, rv_ref,
             w_ref, out_ref):
        dinv = lax.rsqrt(deg_ref[...] + 1.0)
        agg = p_ref[0, :N, :] + p_ref[1, :N, :] + hs_ref[...]
        out1 = agg * dinv + b1_ref[...]
        scale = g_ref[...] * lax.rsqrt(rv_ref[...] + 1e-5)
        h = (out1 - rm_ref[...]) * scale + be_ref[...]
        h = jnp.maximum(h, 0.0)
        h2 = jnp.dot(h, w_ref[...], preferred_element_type=jnp.float32)
        out_ref[...] = h2 * dinv

    return pl.pallas_call(
        body,
        out_shape=jax.ShapeDtypeStruct((N, D), jnp.float32),
    )(deg_col, p1, hs1, b1, gamma, beta, run_mean, run_var, W2)


def _tc_head(deg_col, p2, hs2, b2, batch_row, fc1_W, fc1_b, cls_W, cls_b):
    def body(deg_ref, p_ref, hs_ref, b2_ref, batch_ref, fw_ref, fb_ref,
             cw_ref, cb_ref, out_ref):
        dinv = lax.rsqrt(deg_ref[...] + 1.0)
        agg = p_ref[0, :N, :] + p_ref[1, :N, :] + hs_ref[...]
        out2 = agg * dinv + b2_ref[...]
        seg = lax.broadcasted_iota(jnp.int32, (G, N), 0)
        onehot = (seg == batch_ref[...]).astype(jnp.float32)
        sums = jnp.dot(onehot, out2, preferred_element_type=jnp.float32)
        cnt = jnp.sum(onehot, axis=1, keepdims=True)
        pooled = sums / jnp.maximum(cnt, 1.0)
        h = jnp.dot(pooled, fw_ref[...], preferred_element_type=jnp.float32)
        h = jnp.maximum(h + fb_ref[...], 0.0)
        out_ref[...] = jnp.dot(h, cw_ref[...],
                               preferred_element_type=jnp.float32) + cb_ref[...]

    return pl.pallas_call(
        body,
        out_shape=jax.ShapeDtypeStruct((G, 2), jnp.float32),
    )(deg_col, p2, hs2, b2, batch_row, fc1_W, fc1_b, cls_W, cls_b)


def kernel(x, edge_index, batch, W1, b1, gamma, beta, run_mean, run_var,
           W2, b2, fc1_W, fc1_b, cls_W, cls_b):
    src = edge_index[0]
    dst = edge_index[1]
    pad = E_PAD - E
    packed = src * (MASK + 1) + dst
    packed = jnp.concatenate([packed, jnp.full((pad,), N, packed.dtype)])
    packed = packed.reshape(NW, EPW, CHUNK)

    deg_parts = _sc_degree(packed)
    deg_col = (deg_parts[0, :N] + deg_parts[1, :N]).reshape(N, 1)

    hs1 = _tc_scale_matmul(x, W1, deg_col)
    p1 = _sc_aggregate(hs1, packed)
    hs2 = _tc_mid(deg_col, p1, hs1, b1.reshape(1, D), gamma.reshape(1, D),
                  beta.reshape(1, D), run_mean.reshape(1, D),
                  run_var.reshape(1, D), W2)
    p2 = _sc_aggregate(hs2, packed)
    out = _tc_head(deg_col, p2, hs2, b2.reshape(1, D), batch.reshape(1, N),
                   fc1_W, fc1_b.reshape(1, D // 2), cls_W, cls_b.reshape(1, 2))
    return out
